# Initial kernel scaffold; baseline (speedup 1.0000x reference)
import jax
import jax.numpy as jnp
from jax.experimental import pallas as pl


def kernel(x, y, visible, n_pos, n_neg, lru, memory, params, eps):
    raise NotImplementedError("write your pallas kernel here")


# speedup vs baseline: 1.0160x; 1.0160x over previous
import functools

import jax
import jax.numpy as jnp
from jax.experimental import pallas as pl
from jax.experimental.pallas import tpu as pltpu


def _fused_kernel(scal_ref, x_ref, y_ref, vis_ref, mem_ref,
                  sim_ref, yidx_ref, upd_ref, *, n_pos):
    j = pl.program_id(0)
    xb = x_ref[...]
    mb = mem_ref[...]
    sim = jax.lax.dot_general(xb, mb, (((1,), (1,)), ((), ())))
    sim_ref[...] = sim

    @pl.when(j == 0)
    def _update():
        group_scale = scal_ref[0]
        momentum = scal_ref[3]
        eps = scal_ref[4]
        cols = jax.lax.broadcasted_iota(jnp.int32, (n_pos, n_pos), 1)
        boosted = sim + jnp.where(cols == y_ref[...], 2.0 * group_scale, 0.0)
        y_idx = jnp.argmax(boosted, axis=1).astype(jnp.int32)
        yidx_ref[...] = y_idx[:, None]
        oh = (cols == y_idx[:, None]).astype(jnp.float32)
        get = jax.lax.dot_general(oh, xb, (((0,), (0,)), ((), ())))
        counts = jnp.sum(oh, axis=0)[:, None]
        vis = jnp.max((cols == vis_ref[...]).astype(jnp.float32),
                      axis=0)[:, None]
        valid = jnp.where((counts > 0.1) & (vis > 0.5), 1.0, 0.0)
        keep = valid * momentum + 1.0 - valid
        blend = (1.0 - momentum) * valid / (counts + eps)
        upd = mb * keep + get * blend
        nrm = jnp.maximum(
            jnp.sqrt(jnp.sum(upd * upd, axis=1, keepdims=True)), 1e-12)
        upd_ref[...] = upd / nrm


def kernel(x, y, visible, n_pos, n_neg, lru, memory, params, eps):
    n_pos_static = visible.shape[1]
    M, d = memory.shape
    tile = n_pos_static
    n_tiles = M // tile
    scal = jnp.concatenate(
        [params.astype(jnp.float32),
         jnp.asarray(eps, jnp.float32)[None]])
    x_pos = x[:n_pos_static]
    y2 = y.astype(jnp.int32)[:, None]
    vis2 = visible.astype(jnp.int32).reshape(-1)[None, :]

    grid_spec = pltpu.PrefetchScalarGridSpec(
        num_scalar_prefetch=0,
        grid=(n_tiles,),
        in_specs=[
            pl.BlockSpec(memory_space=pltpu.SMEM),
            pl.BlockSpec((n_pos_static, d), lambda j: (0, 0)),
            pl.BlockSpec((n_pos_static, 1), lambda j: (0, 0)),
            pl.BlockSpec((1, n_pos_static), lambda j: (0, 0)),
            pl.BlockSpec((tile, d), lambda j: (j, 0)),
        ],
        out_specs=[
            pl.BlockSpec((n_pos_static, tile), lambda j: (0, j)),
            pl.BlockSpec((n_pos_static, 1), lambda j: (0, 0)),
            pl.BlockSpec((n_pos_static, d), lambda j: (0, 0)),
        ],
    )
    sim, yidx, upd = pl.pallas_call(
        functools.partial(_fused_kernel, n_pos=n_pos_static),
        grid_spec=grid_spec,
        out_shape=[
            jax.ShapeDtypeStruct((n_pos_static, M), jnp.float32),
            jax.ShapeDtypeStruct((n_pos_static, 1), jnp.int32),
            jax.ShapeDtypeStruct((n_pos_static, d), jnp.float32),
        ],
    )(scal, x_pos, y2, vis2, memory)

    y_idx = yidx.reshape(n_pos_static)
    new_memory = memory.at[:n_pos_static].set(upd)
    start = n_pos + lru * n_neg
    new_memory = jax.lax.dynamic_update_slice(
        new_memory, x[n_pos_static:], (start, 0))
    return (sim, y_idx, new_memory)


# speedup vs baseline: 1.0354x; 1.0191x over previous
import functools

import jax
import jax.numpy as jnp
from jax.experimental import pallas as pl
from jax.experimental.pallas import tpu as pltpu


def _fused_kernel(scal_ref, x_ref, y_ref, vis_ref, mem_ref,
                  sim_ref, yidx_ref, upd_ref, *, n_pos):
    j = pl.program_id(0)
    xb = x_ref[...]
    mb = mem_ref[...]
    sim = jax.lax.dot_general(xb, mb, (((1,), (1,)), ((), ())))
    sim_ref[...] = sim

    @pl.when(j == 0)
    def _update():
        group_scale = scal_ref[0]
        momentum = scal_ref[3]
        eps = scal_ref[4]
        cols = jax.lax.broadcasted_iota(jnp.int32, (n_pos, n_pos), 1)
        boosted = sim[:, :n_pos] + jnp.where(
            cols == y_ref[...], 2.0 * group_scale, 0.0)
        y_idx = jnp.argmax(boosted, axis=1).astype(jnp.int32)
        yidx_ref[...] = y_idx[:, None]
        oh = (cols == y_idx[:, None]).astype(jnp.float32)
        get = jax.lax.dot_general(oh, xb, (((0,), (0,)), ((), ())))
        counts = jnp.sum(oh, axis=0)[:, None]
        vis = jnp.max((cols == vis_ref[...]).astype(jnp.float32),
                      axis=0)[:, None]
        valid = jnp.where((counts > 0.1) & (vis > 0.5), 1.0, 0.0)
        keep = valid * momentum + 1.0 - valid
        blend = (1.0 - momentum) * valid / (counts + eps)
        upd = mb[:n_pos] * keep + get * blend
        nrm = jnp.maximum(
            jnp.sqrt(jnp.sum(upd * upd, axis=1, keepdims=True)), 1e-12)
        upd_ref[...] = upd / nrm


def kernel(x, y, visible, n_pos, n_neg, lru, memory, params, eps):
    n_pos_static = visible.shape[1]
    M, d = memory.shape
    tile = 2048
    n_tiles = M // tile
    scal = jnp.concatenate(
        [params.astype(jnp.float32),
         jnp.asarray(eps, jnp.float32)[None]])
    x_pos = x[:n_pos_static]
    y2 = y.astype(jnp.int32)[:, None]
    vis2 = visible.astype(jnp.int32).reshape(-1)[None, :]

    grid_spec = pltpu.PrefetchScalarGridSpec(
        num_scalar_prefetch=0,
        grid=(n_tiles,),
        in_specs=[
            pl.BlockSpec(memory_space=pltpu.SMEM),
            pl.BlockSpec((n_pos_static, d), lambda j: (0, 0)),
            pl.BlockSpec((n_pos_static, 1), lambda j: (0, 0)),
            pl.BlockSpec((1, n_pos_static), lambda j: (0, 0)),
            pl.BlockSpec((tile, d), lambda j: (j, 0)),
        ],
        out_specs=[
            pl.BlockSpec((n_pos_static, tile), lambda j: (0, j)),
            pl.BlockSpec((n_pos_static, 1), lambda j: (0, 0)),
            pl.BlockSpec((n_pos_static, d), lambda j: (0, 0)),
        ],
    )
    sim, yidx, upd = pl.pallas_call(
        functools.partial(_fused_kernel, n_pos=n_pos_static),
        grid_spec=grid_spec,
        out_shape=[
            jax.ShapeDtypeStruct((n_pos_static, M), jnp.float32),
            jax.ShapeDtypeStruct((n_pos_static, 1), jnp.int32),
            jax.ShapeDtypeStruct((n_pos_static, d), jnp.float32),
        ],
    )(scal, x_pos, y2, vis2, memory)

    y_idx = yidx.reshape(n_pos_static)
    new_memory = memory.at[:n_pos_static].set(upd)
    start = n_pos + lru * n_neg
    new_memory = jax.lax.dynamic_update_slice(
        new_memory, x[n_pos_static:], (start, 0))
    return (sim, y_idx, new_memory)
